# baseline (speedup 1.0000x reference)
import functools
import math

import jax
import jax.numpy as jnp
from jax import lax
from jax.experimental import pallas as pl
from jax.experimental.pallas import tpu as pltpu
from jax.experimental.pallas import tpu_sc as plsc

_D = 64
_SCALE = math.sqrt(_D)
_L = 16
_SP = 17


def _iota17():
    return jnp.arange(0, _L * _SP, _SP, dtype=jnp.int32)


@functools.lru_cache(maxsize=None)
def _build_transpose(v_rows: int):
    info = plsc.get_sparse_core_info()
    nw = info.num_cores * info.num_subcores
    chunk = 256
    n_full = v_rows // chunk
    n_even = (n_full // nw) * nw
    slots = n_even // nw
    rem_full = n_full - n_even
    tail = v_rows - n_full * chunk
    mesh = plsc.VectorSubcoreMesh(core_axis_name="c", subcore_axis_name="s")

    @functools.partial(
        pl.kernel,
        out_type=jax.ShapeDtypeStruct((v_rows, 128), jnp.float32),
        mesh=mesh,
        scratch_types=[
            pltpu.VMEM((_D, chunk), jnp.float32),
            pltpu.VMEM((_D, chunk), jnp.float32),
            pltpu.VMEM((chunk, 128), jnp.float32),
            pltpu.VMEM((chunk, 128), jnp.float32),
            pltpu.VMEM((_L * _SP,), jnp.float32),
            pltpu.VMEM((_L * _SP,), jnp.float32),
            pltpu.SemaphoreType.DMA,
            pltpu.SemaphoreType.DMA,
            pltpu.SemaphoreType.DMA,
            pltpu.SemaphoreType.DMA,
        ],
        compiler_params=pltpu.CompilerParams(needs_layout_passes=False),
    )
    def tr_kernel(src_hbm, tail_hbm, out_hbm, src0, src1, dst0, dst1,
                  stg0, stg1, gs0, gs1, ws0, ws1):
        wid = lax.axis_index("s") * info.num_cores + lax.axis_index("c")
        srcs = (src0, src1)
        dsts = (dst0, dst1)
        stgs = (stg0, stg1)
        gsems = (gs0, gs1)
        wsems = (ws0, ws1)
        i17 = _iota17()

        def chunk_id(slot):
            return wid + nw * slot

        def start_read(slot, b):
            v0 = pl.multiple_of(chunk_id(slot) * chunk, 128)
            pltpu.async_copy(src_hbm.at[:, pl.ds(v0, chunk)], srcs[b],
                             gsems[b])

        def wait_read(b):
            pltpu.make_async_copy(src_hbm.at[:, pl.ds(0, chunk)], srcs[b],
                                  gsems[b]).wait()

        def wait_write(b):
            pltpu.make_async_copy(dsts[b], out_hbm.at[pl.ds(0, chunk)],
                                  wsems[b]).wait()

        def transpose_chunk(b):
            sv = srcs[b]
            dv = dsts[b]

            def g_loop(g, carry):
                t0 = g * _L
                for k in range(4):
                    d0 = k * _L
                    stg = stgs[k % 2]
                    for i in range(_L):
                        vals = sv[d0 + i, pl.ds(t0, _L)]
                        plsc.store_scatter(stg, [i17 + i], vals)
                    for j in range(_L):
                        dv[t0 + j, pl.ds(d0, _L)] = stg[pl.ds(j * _SP, _L)]
                return carry

            lax.fori_loop(0, chunk // _L, g_loop, 0)

        def start_write(slot, b):
            j0 = pl.multiple_of(chunk_id(slot) * chunk, 8)
            pltpu.async_copy(dsts[b], out_hbm.at[pl.ds(j0, chunk)],
                             wsems[b])

        start_read(0, 0)
        start_read(1, 1)
        wait_read(0)
        transpose_chunk(0)
        start_write(0, 0)
        start_read(2, 0)
        wait_read(1)
        transpose_chunk(1)
        start_write(1, 1)
        start_read(3, 1)

        def pair_body(i, carry):
            g = 2 * i
            for b in range(2):
                wait_read(b)
                wait_write(b)
                transpose_chunk(b)
                start_write(g + b, b)
                start_read(g + b + 2, b)
            return carry

        lax.fori_loop(1, slots // 2 - 1, pair_body, 0)
        g = slots - 2
        for b in range(2):
            wait_read(b)
            wait_write(b)
            transpose_chunk(b)
            start_write(g + b, b)
        wait_write(0)
        wait_write(1)

        @pl.when(wid == 0)
        def _tail():
            for k in range(rem_full):
                ci = n_even + k
                v0 = ci * chunk
                pltpu.sync_copy(src_hbm.at[:, pl.ds(v0, chunk)], src0)
                transpose_chunk(0)
                pltpu.sync_copy(dst0, out_hbm.at[pl.ds(v0, chunk)])
            if tail:
                pltpu.sync_copy(tail_hbm, dst0.at[pl.ds(0, tail)])
                pltpu.sync_copy(dst0.at[pl.ds(0, tail)],
                                out_hbm.at[pl.ds(v_rows - tail, tail)])

    return tr_kernel


@functools.lru_cache(maxsize=None)
def _build_gather(n_b: int, n_s: int):
    info = plsc.get_sparse_core_info()
    nw = info.num_cores * info.num_subcores
    bc = 128
    assert n_s % 8 == 0 and n_b % bc == 0
    n_bchunks = n_b // bc
    n_units = (n_s // 8) * n_bchunks
    assert n_units % nw == 0
    units_per_w = n_units // nw
    n_groups = bc // _L
    mesh = plsc.VectorSubcoreMesh(core_axis_name="c", subcore_axis_name="s")

    @functools.partial(
        pl.kernel,
        out_type=jax.ShapeDtypeStruct((n_s, _D, n_b), jnp.float32),
        mesh=mesh,
        scratch_types=[
            pltpu.VMEM((8, bc), jnp.int32),
            pltpu.VMEM((bc,), jnp.int32),
            pltpu.VMEM((bc,), jnp.int32),
            pltpu.VMEM((bc, 128), jnp.float32),
            pltpu.VMEM((bc, 128), jnp.float32),
            pltpu.VMEM((_D, bc), jnp.float32),
            pltpu.VMEM((_D, bc), jnp.float32),
            pltpu.VMEM((_L * _SP,), jnp.float32),
            pltpu.VMEM((_L * _SP,), jnp.float32),
            pltpu.SemaphoreType.DMA,
            pltpu.SemaphoreType.DMA,
            pltpu.SemaphoreType.DMA,
            pltpu.SemaphoreType.DMA,
        ],
        compiler_params=pltpu.CompilerParams(needs_layout_passes=False),
    )
    def g_kernel(xt_hbm, tbl_hbm, out_hbm, idx_v, j0_v, j1_v,
                 rows0, rows1, blk0, blk1, stg0, stg1,
                 gs0, gs1, ws0, ws1):
        wid = lax.axis_index("s") * info.num_cores + lax.axis_index("c")
        jbufs = (j0_v, j1_v)
        rowss = (rows0, rows1)
        blks = (blk0, blk1)
        stgs = (stg0, stg1)
        gsems = (gs0, gs1)
        wsems = (ws0, ws1)
        i17 = _iota17()

        def load_idx(u):
            oct8 = pl.multiple_of((u // n_bchunks) * 8, 8)
            b0 = pl.multiple_of((u % n_bchunks) * bc, bc)
            pltpu.sync_copy(xt_hbm.at[pl.ds(oct8, 8), pl.ds(b0, bc)],
                            idx_v)

        def start_gather(r, b):
            jb = jbufs[b]
            for g in range(n_groups):
                s = pl.ds(g * _L, _L)
                jb[s] = idx_v[r, s]
            pltpu.async_copy(tbl_hbm.at[jb], rowss[b], gsems[b])

        def finish(u, r, b, wait_wb):
            oct_ = u // n_bchunks
            b0 = pl.multiple_of((u % n_bchunks) * bc, bc)
            pltpu.make_async_copy(tbl_hbm.at[jbufs[b]], rowss[b],
                                  gsems[b]).wait()
            rv = rowss[b]
            bv = blks[b]
            if wait_wb:
                pltpu.make_async_copy(bv, out_hbm.at[0, :, pl.ds(0, bc)],
                                      wsems[b]).wait()

            def t_loop(g, carry):
                t0 = g * _L
                for k in range(4):
                    d0 = k * _L
                    stg = stgs[k % 2]
                    for i in range(_L):
                        vals = rv[t0 + i, pl.ds(d0, _L)]
                        plsc.store_scatter(stg, [i17 + i], vals * _SCALE)
                    for j in range(_L):
                        bv[d0 + j, pl.ds(t0, _L)] = stg[pl.ds(j * _SP, _L)]
                return carry

            lax.fori_loop(0, bc // _L, t_loop, 0)
            pltpu.async_copy(bv, out_hbm.at[oct_ * 8 + r, :,
                                            pl.ds(b0, bc)],
                             wsems[b])

        def run_unit(u, first):
            load_idx(u)
            start_gather(0, 0)
            for r in range(8):
                b = r % 2
                if r + 1 < 8:
                    start_gather(r + 1, 1 - b)
                finish(u, r, b, wait_wb=(not first) or r >= 2)

        run_unit(wid * units_per_w, True)

        def unit_body(i, carry):
            u = wid * units_per_w + i
            load_idx(u)
            start_gather(0, 0)
            for r in range(8):
                b = r % 2
                if r + 1 < 8:
                    start_gather(r + 1, 1 - b)
                finish(u, r, b, True)
            return carry

        lax.fori_loop(1, units_per_w, unit_body, 0)
        pltpu.make_async_copy(blk0, out_hbm.at[0, :, pl.ds(0, bc)],
                              ws0).wait()
        pltpu.make_async_copy(blk1, out_hbm.at[0, :, pl.ds(0, bc)],
                              ws1).wait()

    return g_kernel


def kernel(x, table):
    n_b, n_s = x.shape
    v_rows = table.shape[0]
    xt = jnp.transpose(x)
    tt = jnp.transpose(table)
    t0 = (v_rows // 256) * 256
    tail2 = jnp.pad(table[t0:], ((0, 0), (0, 64)))
    packed = _build_transpose(v_rows)(tt, tail2)
    out = _build_gather(n_b, n_s)(xt, packed)
    return jnp.transpose(out, (2, 0, 1))


# speedup vs baseline: 1.7270x; 1.7270x over previous
import functools
import math

import jax
import jax.numpy as jnp
from jax import lax
from jax.experimental import pallas as pl
from jax.experimental.pallas import tpu as pltpu
from jax.experimental.pallas import tpu_sc as plsc

_D_MODEL = 64
_SCALE = math.sqrt(_D_MODEL)
_LANES = 16


@functools.lru_cache(maxsize=None)
def _build(n_rows: int, d: int, chunk: int):
    info = plsc.get_sparse_core_info()
    nw = info.num_cores * info.num_subcores
    assert n_rows % (nw * chunk) == 0
    b_per_w = n_rows // nw
    n_chunks = b_per_w // chunk
    assert n_chunks % 2 == 0 and n_chunks >= 4
    mesh = plsc.VectorSubcoreMesh(core_axis_name="c", subcore_axis_name="s")

    @functools.partial(
        pl.kernel,
        out_type=jax.ShapeDtypeStruct((n_rows, 2 * d), jnp.float32),
        mesh=mesh,
        scratch_types=[
            pltpu.VMEM((chunk,), jnp.int32),
            pltpu.VMEM((chunk,), jnp.int32),
            pltpu.VMEM((chunk, d), jnp.float32),
            pltpu.VMEM((chunk, d), jnp.float32),
            pltpu.VMEM((chunk, 2 * d), jnp.float32),
            pltpu.VMEM((chunk, 2 * d), jnp.float32),
            pltpu.SemaphoreType.DMA,
            pltpu.SemaphoreType.DMA,
        ],
        compiler_params=pltpu.CompilerParams(use_tc_tiling_on_sc=False),
    )
    def emb_kernel(idx_hbm, table_hbm, out_hbm, idx0, idx1, rows0, rows1,
                   cv0, cv1, sem0, sem1):
        wid = lax.axis_index("s") * info.num_cores + lax.axis_index("c")
        base = wid * b_per_w
        idxs = (idx0, idx1)
        rowss = (rows0, rows1)
        cvs = (cv0, cv1)
        sems = (sem0, sem1)

        def start_gather(ci, b):
            off = base + ci * chunk
            pltpu.sync_copy(idx_hbm.at[pl.ds(off, chunk)], idxs[b])
            pltpu.async_copy(table_hbm.at[idxs[b]], rowss[b], sems[b])

        def finish(ci, b):
            pltpu.make_async_copy(table_hbm.at[idxs[b]], rowss[b],
                                  sems[b]).wait()
            rv = rowss[b]
            cv = cvs[b]

            @plsc.parallel_loop(0, chunk, unroll=8)
            def row_body(r):
                for j in range(d // _LANES):
                    s = pl.ds(j * _LANES, _LANES)
                    cv[r, s] = rv[r, s] * _SCALE
            off = base + ci * chunk
            pltpu.sync_copy(cv, out_hbm.at[pl.ds(off, chunk)])

        start_gather(0, 0)

        def pair_body(i, carry):
            g = 2 * i
            start_gather(g + 1, 1)
            finish(g, 0)
            start_gather(g + 2, 0)
            finish(g + 1, 1)
            return carry

        lax.fori_loop(0, n_chunks // 2 - 1, pair_body, 0)
        g = n_chunks - 2
        start_gather(g + 1, 1)
        finish(g, 0)
        finish(g + 1, 1)

    return emb_kernel


def kernel(x, table):
    b, s = x.shape
    d = table.shape[1]
    n_rows = b * s
    flat_idx = x.reshape(n_rows)
    out = _build(n_rows, d, 256)(flat_idx, table)
    return out[:, :d].reshape(b, s, d)


# speedup vs baseline: 1.7522x; 1.0146x over previous
import functools
import math

import jax
import jax.numpy as jnp
from jax import lax
from jax.experimental import pallas as pl
from jax.experimental.pallas import tpu as pltpu
from jax.experimental.pallas import tpu_sc as plsc

_D_MODEL = 64
_SCALE = math.sqrt(_D_MODEL)
_LANES = 16


@functools.lru_cache(maxsize=None)
def _build(n_rows: int, d: int, chunk: int):
    info = plsc.get_sparse_core_info()
    nw = info.num_cores * info.num_subcores
    assert n_rows % (nw * chunk) == 0
    b_per_w = n_rows // nw
    n_chunks = b_per_w // chunk
    assert n_chunks % 2 == 0 and n_chunks >= 4
    mesh = plsc.VectorSubcoreMesh(core_axis_name="c", subcore_axis_name="s")

    @functools.partial(
        pl.kernel,
        out_type=jax.ShapeDtypeStruct((n_rows, 2 * d), jnp.float32),
        mesh=mesh,
        scratch_types=[
            pltpu.VMEM((chunk,), jnp.int32),
            pltpu.VMEM((chunk,), jnp.int32),
            pltpu.VMEM((chunk, d), jnp.float32),
            pltpu.VMEM((chunk, d), jnp.float32),
            pltpu.VMEM((chunk, 2 * d), jnp.float32),
            pltpu.VMEM((chunk, 2 * d), jnp.float32),
            pltpu.SemaphoreType.DMA,
            pltpu.SemaphoreType.DMA,
        ],
        compiler_params=pltpu.CompilerParams(use_tc_tiling_on_sc=False),
    )
    def emb_kernel(idx_hbm, table_hbm, out_hbm, idx0, idx1, rows0, rows1,
                   cv0, cv1, sem0, sem1):
        wid = lax.axis_index("s") * info.num_cores + lax.axis_index("c")
        base = wid * b_per_w
        idxs = (idx0, idx1)
        rowss = (rows0, rows1)
        cvs = (cv0, cv1)
        sems = (sem0, sem1)

        def start_gather(ci, b):
            off = base + ci * chunk
            pltpu.sync_copy(idx_hbm.at[pl.ds(off, chunk)], idxs[b])
            pltpu.async_copy(table_hbm.at[idxs[b]], rowss[b], sems[b])

        def finish(ci, b):
            pltpu.make_async_copy(table_hbm.at[idxs[b]], rowss[b],
                                  sems[b]).wait()
            rv = rowss[b]
            cv = cvs[b]

            @plsc.parallel_loop(0, chunk, unroll=8)
            def row_body(r):
                for j in range(d // _LANES):
                    s = pl.ds(j * _LANES, _LANES)
                    cv[r, s] = rv[r, s] * _SCALE
            off = base + ci * chunk
            pltpu.sync_copy(cv, out_hbm.at[pl.ds(off, chunk)])

        start_gather(0, 0)

        def pair_body(i, carry):
            g = 2 * i
            start_gather(g + 1, 1)
            finish(g, 0)
            start_gather(g + 2, 0)
            finish(g + 1, 1)
            return carry

        lax.fori_loop(0, n_chunks // 2 - 1, pair_body, 0)
        g = n_chunks - 2
        start_gather(g + 1, 1)
        finish(g, 0)
        finish(g + 1, 1)

    return emb_kernel


def kernel(x, table):
    b, s = x.shape
    d = table.shape[1]
    n_rows = b * s
    flat_idx = x.reshape(n_rows)
    out = _build(n_rows, d, 320)(flat_idx, table)
    return out[:, :d].reshape(b, s, d)


# speedup vs baseline: 1.7708x; 1.0106x over previous
import functools
import math

import jax
import jax.numpy as jnp
from jax import lax
from jax.experimental import pallas as pl
from jax.experimental.pallas import tpu as pltpu
from jax.experimental.pallas import tpu_sc as plsc

_D_MODEL = 64
_SCALE = math.sqrt(_D_MODEL)
_LANES = 16


@functools.lru_cache(maxsize=None)
def _build(n_rows: int, d: int, chunk: int):
    info = plsc.get_sparse_core_info()
    nw = info.num_cores * info.num_subcores
    assert n_rows % (nw * chunk) == 0
    b_per_w = n_rows // nw
    n_chunks = b_per_w // chunk
    assert n_chunks % 2 == 0 and n_chunks >= 4
    mesh = plsc.VectorSubcoreMesh(core_axis_name="c", subcore_axis_name="s")

    @functools.partial(
        pl.kernel,
        out_type=jax.ShapeDtypeStruct((n_rows, 2 * d), jnp.float32),
        mesh=mesh,
        scratch_types=[
            pltpu.VMEM((chunk,), jnp.int32),
            pltpu.VMEM((chunk,), jnp.int32),
            pltpu.VMEM((chunk, d), jnp.float32),
            pltpu.VMEM((chunk, d), jnp.float32),
            pltpu.VMEM((chunk, 2 * d), jnp.float32),
            pltpu.SemaphoreType.DMA,
            pltpu.SemaphoreType.DMA,
        ],
        compiler_params=pltpu.CompilerParams(use_tc_tiling_on_sc=False),
    )
    def emb_kernel(idx_hbm, table_hbm, out_hbm, idx0, idx1, rows0, rows1,
                   cv0, sem0, sem1):
        wid = lax.axis_index("s") * info.num_cores + lax.axis_index("c")
        base = wid * b_per_w
        idxs = (idx0, idx1)
        rowss = (rows0, rows1)
        cvs = (cv0, cv0)
        sems = (sem0, sem1)

        def start_gather(ci, b):
            off = base + ci * chunk
            pltpu.sync_copy(idx_hbm.at[pl.ds(off, chunk)], idxs[b])
            pltpu.async_copy(table_hbm.at[idxs[b]], rowss[b], sems[b])

        def finish(ci, b):
            pltpu.make_async_copy(table_hbm.at[idxs[b]], rowss[b],
                                  sems[b]).wait()
            rv = rowss[b]
            cv = cvs[b]

            @plsc.parallel_loop(0, chunk, unroll=8)
            def row_body(r):
                for j in range(d // _LANES):
                    s = pl.ds(j * _LANES, _LANES)
                    cv[r, s] = rv[r, s] * _SCALE
            off = base + ci * chunk
            pltpu.sync_copy(cv, out_hbm.at[pl.ds(off, chunk)])

        start_gather(0, 0)

        def pair_body(i, carry):
            g = 2 * i
            start_gather(g + 1, 1)
            finish(g, 0)
            start_gather(g + 2, 0)
            finish(g + 1, 1)
            return carry

        lax.fori_loop(0, n_chunks // 2 - 1, pair_body, 0)
        g = n_chunks - 2
        start_gather(g + 1, 1)
        finish(g, 0)
        finish(g + 1, 1)

    return emb_kernel


def kernel(x, table):
    b, s = x.shape
    d = table.shape[1]
    n_rows = b * s
    flat_idx = x.reshape(n_rows)
    out = _build(n_rows, d, 400)(flat_idx, table)
    return out[:, :d].reshape(b, s, d)
